# Initial kernel scaffold; baseline (speedup 1.0000x reference)
import jax
import jax.numpy as jnp
from jax.experimental import pallas as pl


def kernel(distances, table):
    raise NotImplementedError("write your pallas kernel here")


# speedup vs baseline: 10.5868x; 10.5868x over previous
import jax
import jax.numpy as jnp
from jax import lax
from jax.experimental import pallas as pl
from jax.experimental.pallas import tpu as pltpu
from jax.experimental.pallas import tpu_sc as plsc

NC, NS, L = 2, 16, 16
B, N, H = 32, 256, 16
E = N * N
CHUNK = 2048
VECS = CHUNK // L
NCHUNKS = E // CHUNK


def _body(d_hbm, tbl_hbm, out_hbm, tbl_v, idx_v, out_v):
    wid = lax.axis_index("s") * NC + lax.axis_index("c")
    pltpu.sync_copy(tbl_hbm, tbl_v)

    def chunk_body(c, carry):
        pltpu.sync_copy(d_hbm.at[wid, pl.ds(c * CHUNK, CHUNK)], idx_v)

        def vec_body(v, carry2):
            idx16 = idx_v[pl.ds(v * L, L)] << 4
            for h in range(H):
                out_v[h, pl.ds(v * L, L)] = plsc.load_gather(tbl_v, [idx16 + h])
            return carry2

        lax.fori_loop(0, VECS, vec_body, 0)
        pltpu.sync_copy(out_v, out_hbm.at[wid, :, pl.ds(c * CHUNK, CHUNK)])
        return carry

    lax.fori_loop(0, NCHUNKS, chunk_body, 0)


def kernel(distances, table):
    d = distances.astype(jnp.int32).reshape(B, E)
    tbl = jnp.zeros((L, H), jnp.float32).at[: table.shape[0]].set(table).reshape(L * H)
    mesh = plsc.VectorSubcoreMesh(
        core_axis_name="c", subcore_axis_name="s", num_cores=NC, num_subcores=NS
    )
    out = pl.kernel(
        _body,
        out_type=jax.ShapeDtypeStruct((B, H, E), jnp.float32),
        mesh=mesh,
        compiler_params=pltpu.CompilerParams(needs_layout_passes=False),
        scratch_types=[
            pltpu.VMEM((L * H,), jnp.float32),
            pltpu.VMEM((CHUNK,), jnp.int32),
            pltpu.VMEM((H, CHUNK), jnp.float32),
        ],
    )(d, tbl)
    return out.reshape(B, H, N, N)


# speedup vs baseline: 24.2771x; 2.2932x over previous
import jax
import jax.numpy as jnp
from jax import lax
from jax.experimental import pallas as pl
from jax.experimental.pallas import tpu as pltpu
from jax.experimental.pallas import tpu_sc as plsc

NC, NS, L = 2, 16, 16
B, N, H = 32, 256, 16
E = N * N
CHUNK = 2048
VECS = CHUNK // L
NCHUNKS = E // CHUNK
NPAIRS = NCHUNKS // 2


def _body(d_hbm, tbl_hbm, out_hbm, tbl_v, idx0, idx1, out0, out1,
          sin0, sin1, sout0, sout1):
    wid = lax.axis_index("s") * NC + lax.axis_index("c")
    pltpu.sync_copy(tbl_hbm, tbl_v)

    ibufs, obufs = (idx0, idx1), (out0, out1)
    sins, souts = (sin0, sin1), (sout0, sout1)

    def in_slice(c):
        return d_hbm.at[wid, pl.ds(c * CHUNK, CHUNK)]

    def out_slice(c):
        return out_hbm.at[wid, :, pl.ds(c * CHUNK, CHUNK)]

    def start_load(c, b):
        pltpu.async_copy(in_slice(c), ibufs[b], sins[b])

    def wait_load(c, b):
        pltpu.make_async_copy(in_slice(c), ibufs[b], sins[b]).wait()

    def start_store(c, b):
        pltpu.async_copy(obufs[b], out_slice(c), souts[b])

    def wait_store(c, b):
        pltpu.make_async_copy(obufs[b], out_slice(c), souts[b]).wait()

    def compute(b):
        ibuf, obuf = ibufs[b], obufs[b]

        @plsc.parallel_loop(0, VECS, 1, unroll=2)
        def vec_body(v):
            idx16 = ibuf[pl.ds(v * L, L)] << 4
            vals = [plsc.load_gather(tbl_v, [idx16 + h if h else idx16])
                    for h in range(H)]
            for h in range(H):
                obuf[h, pl.ds(v * L, L)] = vals[h]

    start_load(0, 0)

    def pair(p, carry):
        c0 = p * 2
        wait_load(c0, 0)
        start_load(c0 + 1, 1)

        @pl.when(p > 0)
        def _():
            wait_store(c0 - 2, 0)

        compute(0)
        start_store(c0, 0)

        wait_load(c0 + 1, 1)

        @pl.when(p > 0)
        def _():
            wait_store(c0 - 1, 1)

        compute(1)
        start_store(c0 + 1, 1)

        @pl.when(p < NPAIRS - 1)
        def _():
            start_load(c0 + 2, 0)

        return carry

    lax.fori_loop(0, NPAIRS, pair, 0)
    wait_store(NCHUNKS - 2, 0)
    wait_store(NCHUNKS - 1, 1)


def kernel(distances, table):
    d = distances.astype(jnp.int32).reshape(B, E)
    tbl = jnp.zeros((L, H), jnp.float32).at[: table.shape[0]].set(table).reshape(L * H)
    mesh = plsc.VectorSubcoreMesh(
        core_axis_name="c", subcore_axis_name="s", num_cores=NC, num_subcores=NS
    )
    out = pl.kernel(
        _body,
        out_type=jax.ShapeDtypeStruct((B, H, E), jnp.float32),
        mesh=mesh,
        compiler_params=pltpu.CompilerParams(needs_layout_passes=False),
        scratch_types=[
            pltpu.VMEM((L * H,), jnp.float32),
            pltpu.VMEM((CHUNK,), jnp.int32),
            pltpu.VMEM((CHUNK,), jnp.int32),
            pltpu.VMEM((H, CHUNK), jnp.float32),
            pltpu.VMEM((H, CHUNK), jnp.float32),
            pltpu.SemaphoreType.DMA,
            pltpu.SemaphoreType.DMA,
            pltpu.SemaphoreType.DMA,
            pltpu.SemaphoreType.DMA,
        ],
    )(d, tbl)
    return out.reshape(B, H, N, N)


# speedup vs baseline: 24.3385x; 1.0025x over previous
import jax
import jax.numpy as jnp
from jax import lax
from jax.experimental import pallas as pl
from jax.experimental.pallas import tpu as pltpu
from jax.experimental.pallas import tpu_sc as plsc

NC, NS, L = 2, 16, 16
B, N, H = 32, 256, 16
E = N * N
CHUNK = 2048
VECS = CHUNK // L
NCHUNKS = E // CHUNK
NPAIRS = NCHUNKS // 2


def _body(d_hbm, tbl_hbm, out_hbm, tbl_v, idx0, idx1, out0, out1,
          sin0, sin1, sout0, sout1):
    wid = lax.axis_index("s") * NC + lax.axis_index("c")
    pltpu.sync_copy(tbl_hbm, tbl_v)

    ibufs, obufs = (idx0, idx1), (out0, out1)
    sins, souts = (sin0, sin1), (sout0, sout1)

    def in_slice(c):
        return d_hbm.at[wid, pl.ds(c * CHUNK, CHUNK)]

    def out_slice(c):
        return out_hbm.at[wid, :, pl.ds(c * CHUNK, CHUNK)]

    def start_load(c, b):
        pltpu.async_copy(in_slice(c), ibufs[b], sins[b])

    def wait_load(c, b):
        pltpu.make_async_copy(in_slice(c), ibufs[b], sins[b]).wait()

    def start_store(c, b):
        pltpu.async_copy(obufs[b], out_slice(c), souts[b])

    def wait_store(c, b):
        pltpu.make_async_copy(obufs[b], out_slice(c), souts[b]).wait()

    def compute(b):
        ibuf, obuf = ibufs[b], obufs[b]

        @plsc.parallel_loop(0, VECS, 1, unroll=2)
        def vec_body(v):
            idx16 = ibuf[pl.ds(v * L, L)] << 4
            vals = [plsc.load_gather(tbl_v, [idx16 + h if h else idx16])
                    for h in range(H)]
            for h in range(H):
                obuf[h, pl.ds(v * L, L)] = vals[h]

    start_load(0, 0)

    def pair(p, carry):
        c0 = p * 2
        wait_load(c0, 0)
        start_load(c0 + 1, 1)

        @pl.when(p > 0)
        def _():
            wait_store(c0 - 2, 0)

        compute(0)
        start_store(c0, 0)

        wait_load(c0 + 1, 1)

        @pl.when(p > 0)
        def _():
            wait_store(c0 - 1, 1)

        compute(1)
        start_store(c0 + 1, 1)

        @pl.when(p < NPAIRS - 1)
        def _():
            start_load(c0 + 2, 0)

        return carry

    lax.fori_loop(0, NPAIRS, pair, 0)
    wait_store(NCHUNKS - 2, 0)
    wait_store(NCHUNKS - 1, 1)


def kernel(distances, table):
    d = distances.astype(jnp.int32).reshape(B, E)
    tbl = jnp.zeros((L, H), jnp.float32).at[: table.shape[0]].set(table).reshape(L * H)
    mesh = plsc.VectorSubcoreMesh(
        core_axis_name="c", subcore_axis_name="s", num_cores=NC, num_subcores=NS
    )
    out = pl.kernel(
        _body,
        out_type=jax.ShapeDtypeStruct((B, H, E), jnp.float32),
        mesh=mesh,
        compiler_params=pltpu.CompilerParams(
            needs_layout_passes=False, use_tc_tiling_on_sc=True
        ),
        scratch_types=[
            pltpu.VMEM((L * H,), jnp.float32),
            pltpu.VMEM((CHUNK,), jnp.int32),
            pltpu.VMEM((CHUNK,), jnp.int32),
            pltpu.VMEM((H, CHUNK), jnp.float32),
            pltpu.VMEM((H, CHUNK), jnp.float32),
            pltpu.SemaphoreType.DMA,
            pltpu.SemaphoreType.DMA,
            pltpu.SemaphoreType.DMA,
            pltpu.SemaphoreType.DMA,
        ],
    )(d, tbl)
    return out.reshape(B, H, N, N)


# speedup vs baseline: 31.8602x; 1.3090x over previous
import jax
import jax.numpy as jnp
from jax import lax
from jax.experimental import pallas as pl
from jax.experimental.pallas import tpu as pltpu
from jax.experimental.pallas import tpu_sc as plsc

NC, NS, L = 2, 16, 16
B, N, H = 32, 256, 16
TS, TL = 8, 128
NTR, NTC = N // TS, N // TL
NCHUNKS = NTR * NTC
VECS = TS * TL // L
NPAIRS = NCHUNKS // 2


def _body(d_hbm, tbl_hbm, out_hbm, tbl_v, idx0, idx1, out0, out1,
          sin0, sin1, sout0, sout1):
    wid = lax.axis_index("s") * NC + lax.axis_index("c")
    pltpu.sync_copy(tbl_hbm, tbl_v)

    ibufs, obufs = (idx0, idx1), (out0, out1)
    sins, souts = (sin0, sin1), (sout0, sout1)

    def in_slice(c):
        tr, tc = c // NTC, c % NTC
        return d_hbm.at[wid, pl.ds(tr * TS, TS), pl.ds(tc * TL, TL)]

    def out_slice(c):
        tr, tc = c // NTC, c % NTC
        return out_hbm.at[wid, :, pl.ds(tr * TS, TS), pl.ds(tc * TL, TL)]

    def start_load(c, b):
        pltpu.async_copy(in_slice(c), ibufs[b], sins[b])

    def wait_load(c, b):
        pltpu.make_async_copy(in_slice(c), ibufs[b], sins[b]).wait()

    def start_store(c, b):
        pltpu.async_copy(obufs[b], out_slice(c), souts[b])

    def wait_store(c, b):
        pltpu.make_async_copy(obufs[b], out_slice(c), souts[b]).wait()

    def compute(b):
        ibuf, obuf = ibufs[b], obufs[b]

        @plsc.parallel_loop(0, VECS, 1, unroll=2)
        def vec_body(v):
            s, g = v >> 3, (v & 7) * L
            idx16 = ibuf[s, pl.ds(g, L)] << 4
            vals = [plsc.load_gather(tbl_v, [idx16 + h if h else idx16])
                    for h in range(H)]
            for h in range(H):
                obuf[h, s, pl.ds(g, L)] = vals[h]

    start_load(0, 0)

    def pair(p, carry):
        c0 = p * 2
        wait_load(c0, 0)
        start_load(c0 + 1, 1)

        @pl.when(p > 0)
        def _():
            wait_store(c0 - 2, 0)

        compute(0)
        start_store(c0, 0)

        wait_load(c0 + 1, 1)

        @pl.when(p > 0)
        def _():
            wait_store(c0 - 1, 1)

        compute(1)
        start_store(c0 + 1, 1)

        @pl.when(p < NPAIRS - 1)
        def _():
            start_load(c0 + 2, 0)

        return carry

    lax.fori_loop(0, NPAIRS, pair, 0)
    wait_store(NCHUNKS - 2, 0)
    wait_store(NCHUNKS - 1, 1)


def kernel(distances, table):
    d = distances.astype(jnp.int32)
    tbl = jnp.zeros((L, H), jnp.float32).at[: table.shape[0]].set(table).reshape(L * H)
    mesh = plsc.VectorSubcoreMesh(
        core_axis_name="c", subcore_axis_name="s", num_cores=NC, num_subcores=NS
    )
    return pl.kernel(
        _body,
        out_type=jax.ShapeDtypeStruct((B, H, N, N), jnp.float32),
        mesh=mesh,
        compiler_params=pltpu.CompilerParams(
            needs_layout_passes=False, use_tc_tiling_on_sc=True
        ),
        scratch_types=[
            pltpu.VMEM((L * H,), jnp.float32),
            pltpu.VMEM((TS, TL), jnp.int32),
            pltpu.VMEM((TS, TL), jnp.int32),
            pltpu.VMEM((H, TS, TL), jnp.float32),
            pltpu.VMEM((H, TS, TL), jnp.float32),
            pltpu.SemaphoreType.DMA,
            pltpu.SemaphoreType.DMA,
            pltpu.SemaphoreType.DMA,
            pltpu.SemaphoreType.DMA,
        ],
    )(d, tbl)


# speedup vs baseline: 34.1048x; 1.0705x over previous
import jax
import jax.numpy as jnp
from jax import lax
from jax.experimental import pallas as pl
from jax.experimental.pallas import tpu as pltpu
from jax.experimental.pallas import tpu_sc as plsc

NC, NS, L = 2, 16, 16
B, N, H = 32, 256, 16
TS, TL = 8, 128
NTC = N // TL
NCHUNKS = N // TS
VECS = TS * TL // L
NPAIRS = NCHUNKS // 2


def _body(d_hbm, tbl_hbm, out_hbm, tbl_v, idx0, idx1, out0, out1,
          sin0, sin1, sout0, sout1):
    wid = lax.axis_index("s") * NC + lax.axis_index("c")
    pltpu.sync_copy(tbl_hbm, tbl_v)

    ibufs, obufs = (idx0, idx1), (out0, out1)
    sins, souts = (sin0, sin1), (sout0, sout1)

    def in_slice(c, tc):
        return d_hbm.at[wid, pl.ds(c * TS, TS), pl.ds(tc * TL, TL)]

    def out_slice(c, tc):
        return out_hbm.at[wid, :, pl.ds(c * TS, TS), pl.ds(tc * TL, TL)]

    def start_load(c, b):
        for tc in range(NTC):
            pltpu.async_copy(in_slice(c, tc), ibufs[b].at[tc], sins[b])

    def wait_load(c, b):
        for tc in range(NTC):
            pltpu.make_async_copy(in_slice(c, tc), ibufs[b].at[tc], sins[b]).wait()

    def start_store(c, b):
        for tc in range(NTC):
            pltpu.async_copy(obufs[b].at[:, tc], out_slice(c, tc), souts[b])

    def wait_store(c, b):
        for tc in range(NTC):
            pltpu.make_async_copy(obufs[b].at[:, tc], out_slice(c, tc), souts[b]).wait()

    def compute(b):
        ibuf, obuf = ibufs[b], obufs[b]
        for tc in range(NTC):

            @plsc.parallel_loop(0, VECS, 1, unroll=2)
            def vec_body(v):
                s, g = v >> 3, (v & 7) * L
                idx16 = ibuf[tc, s, pl.ds(g, L)] << 4
                vals = [plsc.load_gather(tbl_v, [idx16 + h if h else idx16])
                        for h in range(H)]
                for h in range(H):
                    obuf[h, tc, s, pl.ds(g, L)] = vals[h]

    start_load(0, 0)
    start_load(1, 1)
    wait_load(0, 0)
    compute(0)
    start_store(0, 0)
    start_load(2, 0)
    wait_load(1, 1)
    compute(1)
    start_store(1, 1)
    start_load(3, 1)

    def pair(p, carry):
        c0 = p * 2
        wait_load(c0, 0)
        wait_store(c0 - 2, 0)
        compute(0)
        start_store(c0, 0)
        start_load(c0 + 2, 0)
        wait_load(c0 + 1, 1)
        wait_store(c0 - 1, 1)
        compute(1)
        start_store(c0 + 1, 1)
        start_load(c0 + 3, 1)
        return carry

    lax.fori_loop(1, NPAIRS - 1, pair, 0)

    c0 = NCHUNKS - 2
    wait_load(c0, 0)
    wait_store(c0 - 2, 0)
    compute(0)
    start_store(c0, 0)
    wait_load(c0 + 1, 1)
    wait_store(c0 - 1, 1)
    compute(1)
    start_store(c0 + 1, 1)
    wait_store(c0, 0)
    wait_store(c0 + 1, 1)


def kernel(distances, table):
    d = distances.astype(jnp.int32)
    tbl = jnp.zeros((L, H), jnp.float32).at[: table.shape[0]].set(table).reshape(L * H)
    mesh = plsc.VectorSubcoreMesh(
        core_axis_name="c", subcore_axis_name="s", num_cores=NC, num_subcores=NS
    )
    return pl.kernel(
        _body,
        out_type=jax.ShapeDtypeStruct((B, H, N, N), jnp.float32),
        mesh=mesh,
        compiler_params=pltpu.CompilerParams(
            needs_layout_passes=False, use_tc_tiling_on_sc=True
        ),
        scratch_types=[
            pltpu.VMEM((L * H,), jnp.float32),
            pltpu.VMEM((NTC, TS, TL), jnp.int32),
            pltpu.VMEM((NTC, TS, TL), jnp.int32),
            pltpu.VMEM((H, NTC, TS, TL), jnp.float32),
            pltpu.VMEM((H, NTC, TS, TL), jnp.float32),
            pltpu.SemaphoreType.DMA,
            pltpu.SemaphoreType.DMA,
            pltpu.SemaphoreType.DMA,
            pltpu.SemaphoreType.DMA,
        ],
    )(d, tbl)


# speedup vs baseline: 80.5212x; 2.3610x over previous
import jax
import jax.numpy as jnp
from jax import lax
from jax.experimental import pallas as pl
from jax.experimental.pallas import tpu as pltpu
from jax.experimental.pallas import tpu_sc as plsc

NC, NS, L = 2, 16, 16
B, N, H = 32, 256, 16
TS, TL = 8, 128
NTC = N // TL
NCHUNKS = N // TS
VECS = TS * TL // L
NPAIRS = NCHUNKS // 2


def _body(d_hbm, tbl_hbm, out_hbm, tbl_v, idx0, idx1, out0, out1,
          sin0, sin1, sout0, sout1):
    wid = lax.axis_index("s") * NC + lax.axis_index("c")
    pltpu.sync_copy(tbl_hbm, tbl_v)

    ibufs, obufs = (idx0, idx1), (out0, out1)
    sins, souts = (sin0, sin1), (sout0, sout1)

    def in_slice(c, tc):
        return d_hbm.at[wid, pl.ds(c * TS, TS), pl.ds(tc * TL, TL)]

    def out_slice(c, tc):
        return out_hbm.at[wid, :, pl.ds(c * TS, TS), pl.ds(tc * TL, TL)]

    def start_load(c, b):
        for tc in range(NTC):
            pltpu.async_copy(in_slice(c, tc), ibufs[b].at[tc], sins[b])

    def wait_load(c, b):
        for tc in range(NTC):
            pltpu.make_async_copy(in_slice(c, tc), ibufs[b].at[tc], sins[b]).wait()

    def start_store(c, b):
        for tc in range(NTC):
            pltpu.async_copy(obufs[b].at[:, tc], out_slice(c, tc), souts[b])

    def wait_store(c, b):
        for tc in range(NTC):
            pltpu.make_async_copy(obufs[b].at[:, tc], out_slice(c, tc), souts[b]).wait()

    def compute(b):
        ibuf, obuf = ibufs[b], obufs[b]
        for tc in range(NTC):

            @plsc.parallel_loop(0, TS, 1, unroll=2)
            def row_body(s):
                for g in range(TL // L):
                    idx = ibuf[tc, s, pl.ds(g * L, L)]
                    vals = [plsc.load_gather(tbl_v.at[h], [idx])
                            for h in range(H)]
                    for h in range(H):
                        obuf[h, tc, s, pl.ds(g * L, L)] = vals[h]

    start_load(0, 0)
    start_load(1, 1)
    wait_load(0, 0)
    compute(0)
    start_store(0, 0)
    start_load(2, 0)
    wait_load(1, 1)
    compute(1)
    start_store(1, 1)
    start_load(3, 1)

    def pair(p, carry):
        c0 = p * 2
        wait_load(c0, 0)
        wait_store(c0 - 2, 0)
        compute(0)
        start_store(c0, 0)
        start_load(c0 + 2, 0)
        wait_load(c0 + 1, 1)
        wait_store(c0 - 1, 1)
        compute(1)
        start_store(c0 + 1, 1)
        start_load(c0 + 3, 1)
        return carry

    lax.fori_loop(1, NPAIRS - 1, pair, 0)

    c0 = NCHUNKS - 2
    wait_load(c0, 0)
    wait_store(c0 - 2, 0)
    compute(0)
    start_store(c0, 0)
    wait_load(c0 + 1, 1)
    wait_store(c0 - 1, 1)
    compute(1)
    start_store(c0 + 1, 1)
    wait_store(c0, 0)
    wait_store(c0 + 1, 1)


def kernel(distances, table):
    d = distances.astype(jnp.int32)
    tbl = jnp.zeros((L, H), jnp.float32).at[: table.shape[0]].set(table).T
    mesh = plsc.VectorSubcoreMesh(
        core_axis_name="c", subcore_axis_name="s", num_cores=NC, num_subcores=NS
    )
    return pl.kernel(
        _body,
        out_type=jax.ShapeDtypeStruct((B, H, N, N), jnp.float32),
        mesh=mesh,
        compiler_params=pltpu.CompilerParams(
            needs_layout_passes=False, use_tc_tiling_on_sc=True
        ),
        scratch_types=[
            pltpu.VMEM((H, L), jnp.float32),
            pltpu.VMEM((NTC, TS, TL), jnp.int32),
            pltpu.VMEM((NTC, TS, TL), jnp.int32),
            pltpu.VMEM((H, NTC, TS, TL), jnp.float32),
            pltpu.VMEM((H, NTC, TS, TL), jnp.float32),
            pltpu.SemaphoreType.DMA,
            pltpu.SemaphoreType.DMA,
            pltpu.SemaphoreType.DMA,
            pltpu.SemaphoreType.DMA,
        ],
    )(d, tbl)


# speedup vs baseline: 99.4854x; 1.2355x over previous
import jax
import jax.numpy as jnp
from jax import lax
from jax.experimental import pallas as pl
from jax.experimental.pallas import tpu as pltpu
from jax.experimental.pallas import tpu_sc as plsc

NC, NS, L = 2, 16, 16
B, N, H = 32, 256, 16
TS, TL = 8, 128
NTC = N // TL
NCHUNKS = N // TS
VECS = TS * TL // L
NPAIRS = NCHUNKS // 2


def _body(d_hbm, tbl_hbm, out_hbm, tbl_v, idx0, idx1, out0, out1,
          sin0, sin1, sout0, sout1):
    wid = lax.axis_index("s") * NC + lax.axis_index("c")
    pltpu.sync_copy(tbl_hbm, tbl_v)

    ibufs, obufs = (idx0, idx1), (out0, out1)
    sins, souts = (sin0, sin1), (sout0, sout1)

    def in_slice(c, tc):
        return d_hbm.at[wid, pl.ds(c * TS, TS), pl.ds(tc * TL, TL)]

    def out_slice(c, tc):
        return out_hbm.at[wid, :, pl.ds(c * TS, TS), pl.ds(tc * TL, TL)]

    def start_load(c, b):
        for tc in range(NTC):
            pltpu.async_copy(in_slice(c, tc), ibufs[b].at[tc], sins[b])

    def wait_load(c, b):
        for tc in range(NTC):
            pltpu.make_async_copy(in_slice(c, tc), ibufs[b].at[tc], sins[b]).wait()

    def start_store(c, b):
        for tc in range(NTC):
            pltpu.async_copy(obufs[b].at[:, tc], out_slice(c, tc), souts[b])

    def wait_store(c, b):
        for tc in range(NTC):
            pltpu.make_async_copy(obufs[b].at[:, tc], out_slice(c, tc), souts[b]).wait()

    tcols = [tbl_v[h] for h in range(0, H, 2)]

    def vperm(col, idx):
        return lax.gather(
            col, idx[:, None],
            lax.GatherDimensionNumbers(
                offset_dims=(), collapsed_slice_dims=(0,), start_index_map=(0,)
            ),
            slice_sizes=(1,),
            mode=lax.GatherScatterMode.PROMISE_IN_BOUNDS,
        )

    def compute(b):
        ibuf, obuf = ibufs[b], obufs[b]
        for tc in range(NTC):

            @plsc.parallel_loop(0, TS, 1, unroll=2)
            def row_body(s):
                for g in range(TL // L):
                    idx = ibuf[tc, s, pl.ds(g * L, L)]
                    vals = [vperm(tcols[h // 2], idx) if h % 2 == 0
                            else plsc.load_gather(tbl_v.at[h], [idx])
                            for h in range(H)]
                    for h in range(H):
                        obuf[h, tc, s, pl.ds(g * L, L)] = vals[h]

    start_load(0, 0)
    start_load(1, 1)
    wait_load(0, 0)
    compute(0)
    start_store(0, 0)
    start_load(2, 0)
    wait_load(1, 1)
    compute(1)
    start_store(1, 1)
    start_load(3, 1)

    def pair(p, carry):
        c0 = p * 2
        wait_load(c0, 0)
        wait_store(c0 - 2, 0)
        compute(0)
        start_store(c0, 0)
        start_load(c0 + 2, 0)
        wait_load(c0 + 1, 1)
        wait_store(c0 - 1, 1)
        compute(1)
        start_store(c0 + 1, 1)
        start_load(c0 + 3, 1)
        return carry

    lax.fori_loop(1, NPAIRS - 1, pair, 0)

    c0 = NCHUNKS - 2
    wait_load(c0, 0)
    wait_store(c0 - 2, 0)
    compute(0)
    start_store(c0, 0)
    wait_load(c0 + 1, 1)
    wait_store(c0 - 1, 1)
    compute(1)
    start_store(c0 + 1, 1)
    wait_store(c0, 0)
    wait_store(c0 + 1, 1)


def kernel(distances, table):
    d = distances.astype(jnp.int32)
    tbl = jnp.zeros((L, H), jnp.float32).at[: table.shape[0]].set(table).T
    mesh = plsc.VectorSubcoreMesh(
        core_axis_name="c", subcore_axis_name="s", num_cores=NC, num_subcores=NS
    )
    return pl.kernel(
        _body,
        out_type=jax.ShapeDtypeStruct((B, H, N, N), jnp.float32),
        mesh=mesh,
        compiler_params=pltpu.CompilerParams(
            needs_layout_passes=False, use_tc_tiling_on_sc=True
        ),
        scratch_types=[
            pltpu.VMEM((H, L), jnp.float32),
            pltpu.VMEM((NTC, TS, TL), jnp.int32),
            pltpu.VMEM((NTC, TS, TL), jnp.int32),
            pltpu.VMEM((H, NTC, TS, TL), jnp.float32),
            pltpu.VMEM((H, NTC, TS, TL), jnp.float32),
            pltpu.SemaphoreType.DMA,
            pltpu.SemaphoreType.DMA,
            pltpu.SemaphoreType.DMA,
            pltpu.SemaphoreType.DMA,
        ],
    )(d, tbl)


# speedup vs baseline: 101.4951x; 1.0202x over previous
import jax
import jax.numpy as jnp
from jax import lax
from jax.experimental import pallas as pl
from jax.experimental.pallas import tpu as pltpu
from jax.experimental.pallas import tpu_sc as plsc

NC, NS, L = 2, 16, 16
B, N, H = 32, 256, 16
TS, TL = 8, 128
NTC = N // TL
NCHUNKS = N // TS
VECS = TS * TL // L
NPAIRS = NCHUNKS // 2


def _body(d_hbm, tbl_hbm, out_hbm, tbl_v, idx0, idx1, out0, out1,
          sin0, sin1, sout0, sout1):
    wid = lax.axis_index("s") * NC + lax.axis_index("c")
    pltpu.sync_copy(tbl_hbm, tbl_v)

    ibufs, obufs = (idx0, idx1), (out0, out1)
    sins, souts = (sin0, sin1), (sout0, sout1)

    def in_slice(c, tc):
        return d_hbm.at[wid, pl.ds(c * TS, TS), pl.ds(tc * TL, TL)]

    def out_slice(c, tc):
        return out_hbm.at[wid, :, pl.ds(c * TS, TS), pl.ds(tc * TL, TL)]

    def start_load(c, b):
        for tc in range(NTC):
            pltpu.async_copy(in_slice(c, tc), ibufs[b].at[tc], sins[b])

    def wait_load(c, b):
        for tc in range(NTC):
            pltpu.make_async_copy(in_slice(c, tc), ibufs[b].at[tc], sins[b]).wait()

    def start_store(c, b):
        for tc in range(NTC):
            pltpu.async_copy(obufs[b].at[tc], out_slice(c, tc), souts[b])

    def wait_store(c, b):
        for tc in range(NTC):
            pltpu.make_async_copy(obufs[b].at[tc], out_slice(c, tc), souts[b]).wait()

    tcols = [tbl_v[h] for h in range(0, H, 2)]

    def vperm(col, idx):
        return lax.gather(
            col, idx[:, None],
            lax.GatherDimensionNumbers(
                offset_dims=(), collapsed_slice_dims=(0,), start_index_map=(0,)
            ),
            slice_sizes=(1,),
            mode=lax.GatherScatterMode.PROMISE_IN_BOUNDS,
        )

    def compute(b):
        ibuf, obuf = ibufs[b], obufs[b]
        for tc in range(NTC):

            @plsc.parallel_loop(0, TS, 1, unroll=2)
            def row_body(s):
                for g in range(TL // L):
                    idx = ibuf[tc, s, pl.ds(g * L, L)]
                    vals = [vperm(tcols[h // 2], idx) if h % 2 == 0
                            else plsc.load_gather(tbl_v.at[h], [idx])
                            for h in range(H)]
                    for h in range(H):
                        obuf[tc, h, s, pl.ds(g * L, L)] = vals[h]

    start_load(0, 0)
    start_load(1, 1)
    wait_load(0, 0)
    compute(0)
    start_store(0, 0)
    start_load(2, 0)
    wait_load(1, 1)
    compute(1)
    start_store(1, 1)
    start_load(3, 1)

    def pair(p, carry):
        c0 = p * 2
        wait_load(c0, 0)
        wait_store(c0 - 2, 0)
        compute(0)
        start_store(c0, 0)
        start_load(c0 + 2, 0)
        wait_load(c0 + 1, 1)
        wait_store(c0 - 1, 1)
        compute(1)
        start_store(c0 + 1, 1)
        start_load(c0 + 3, 1)
        return carry

    lax.fori_loop(1, NPAIRS - 1, pair, 0)

    c0 = NCHUNKS - 2
    wait_load(c0, 0)
    wait_store(c0 - 2, 0)
    compute(0)
    start_store(c0, 0)
    wait_load(c0 + 1, 1)
    wait_store(c0 - 1, 1)
    compute(1)
    start_store(c0 + 1, 1)
    wait_store(c0, 0)
    wait_store(c0 + 1, 1)


def kernel(distances, table):
    d = distances.astype(jnp.int32)
    tbl = jnp.zeros((L, H), jnp.float32).at[: table.shape[0]].set(table).T
    mesh = plsc.VectorSubcoreMesh(
        core_axis_name="c", subcore_axis_name="s", num_cores=NC, num_subcores=NS
    )
    return pl.kernel(
        _body,
        out_type=jax.ShapeDtypeStruct((B, H, N, N), jnp.float32),
        mesh=mesh,
        compiler_params=pltpu.CompilerParams(
            needs_layout_passes=False, use_tc_tiling_on_sc=True
        ),
        scratch_types=[
            pltpu.VMEM((H, L), jnp.float32),
            pltpu.VMEM((NTC, TS, TL), jnp.int32),
            pltpu.VMEM((NTC, TS, TL), jnp.int32),
            pltpu.VMEM((NTC, H, TS, TL), jnp.float32),
            pltpu.VMEM((NTC, H, TS, TL), jnp.float32),
            pltpu.SemaphoreType.DMA,
            pltpu.SemaphoreType.DMA,
            pltpu.SemaphoreType.DMA,
            pltpu.SemaphoreType.DMA,
        ],
    )(d, tbl)
